# baseline (speedup 1.0000x reference)
import jax
import jax.numpy as jnp
from jax import lax
from jax.experimental import pallas as pl
from jax.experimental.pallas import tpu as pltpu
from jax.experimental.pallas import tpu_sc as plsc

B, L, C, D = 4096, 50, 16, 128
WV = 100000
LP = 52
NC, NS = 2, 16
NW = NC * NS
RPW = B // NW
G = 2
GIDS = G * LP
NGRP = RPW // G
SGRP = 8
NSG = RPW // SGRP
LANE = 16

_mesh = plsc.VectorSubcoreMesh(core_axis_name="c", subcore_axis_name="s")
_sc_params = pltpu.CompilerParams(needs_layout_passes=False)


def _worker_id():
    return lax.axis_index("s") * NC + lax.axis_index("c")


NBUF = 4
NOUT = NGRP // NBUF


def _pool_body(wids_hbm, wemb_hbm, out_hbm, ids_v, buf_v, pool_v, *sems):
    w = _worker_id()
    pltpu.sync_copy(wids_hbm.at[pl.ds(w * NGRP, NGRP)], ids_v)

    def fire(j, s):
        pltpu.async_copy(wemb_hbm.at[ids_v.at[j]], buf_v.at[s], sems[s])

    for s in range(NBUF):
        fire(s, s)

    def outer(jj, carry):
        for s in range(NBUF):
            j = jj * NBUF + s
            pltpu.make_async_copy(
                wemb_hbm.at[ids_v.at[j]], buf_v.at[s], sems[s]).wait()

            def accum(r, accs):
                return tuple(
                    accs[t * (D // LANE) + k]
                    + buf_v[s, t * LP + r, pl.ds(k * LANE, LANE)]
                    for t in range(G) for k in range(D // LANE))

            init = tuple(buf_v[s, t * LP, pl.ds(k * LANE, LANE)]
                         for t in range(G) for k in range(D // LANE))
            accs = lax.fori_loop(1, L, accum, init)
            for t in range(G):
                for k in range(D // LANE):
                    pool_v[pl.ds((j * G + t) * D + k * LANE, LANE)] = (
                        accs[t * (D // LANE) + k])

            @pl.when(j + NBUF < NGRP)
            def _():
                fire(j + NBUF, s)
        return carry

    lax.fori_loop(0, NOUT, outer, 0)
    pltpu.sync_copy(pool_v, out_hbm.at[pl.ds(w * RPW * D, RPW * D)])


def _pool_call(wids_grp, word_emb):
    return pl.kernel(
        _pool_body,
        out_type=jax.ShapeDtypeStruct((B * D,), jnp.float32),
        mesh=_mesh,
        scratch_types=[
            pltpu.VMEM((NGRP, GIDS), jnp.int32),
            pltpu.VMEM((NBUF, GIDS, D), jnp.float32),
            pltpu.VMEM((RPW * D,), jnp.float32),
        ] + [pltpu.SemaphoreType.DMA] * NBUF,
        compiler_params=_sc_params,
        name="yamada_pool_sc",
    )(wids_grp, word_emb)


def _proj_body(pooled_ref, ids_ref, w_ref, b_ref, out_ref):
    ids = ids_ref[...]
    cnt = jnp.sum((ids != 0).astype(jnp.float32), axis=1, keepdims=True)
    pooled = pooled_ref[...] / jnp.maximum(cnt, 1.0)
    proj = lax.dot_general(pooled, w_ref[...], (((1,), (1,)), ((), ())),
                           preferred_element_type=jnp.float32)
    out_ref[...] = proj + b_ref[...]


def _proj_call(pooled, wids_pad, w, bias):
    return pl.pallas_call(
        _proj_body,
        out_shape=jax.ShapeDtypeStruct((B, D), jnp.float32),
        name="yamada_proj_tc",
    )(pooled, wids_pad, w, bias)


SBUF = 4


def _score_body(cids_hbm, proj_hbm, eemb_hbm, out_hbm, cid_v, ebuf_v, proj_v,
                sbuf_v, *sems):
    w = _worker_id()
    pltpu.sync_copy(cids_hbm.at[pl.ds(w * NSG, NSG)], cid_v)
    pltpu.sync_copy(proj_hbm.at[pl.ds(w * RPW, RPW)], proj_v)
    lanes = lax.iota(jnp.int32, LANE)

    def fire(g, s):
        pltpu.async_copy(eemb_hbm.at[cid_v.at[g]], ebuf_v.at[s], sems[s])

    for s in range(SBUF):
        fire(s, s)

    def outer(gg, carry):
        for s in range(SBUF):
            g = gg * SBUF + s
            pltpu.make_async_copy(eemb_hbm.at[cid_v.at[g]],
                                  ebuf_v.at[s], sems[s]).wait()

            def trow(t, carry2):
                b = g * SGRP + t
                p = [proj_v[b, pl.ds(k * LANE, LANE)]
                     for k in range(D // LANE)]
                svec = jnp.zeros((LANE,), jnp.float32)
                for c in range(C):
                    er = t * C + c
                    a0 = p[0] * ebuf_v[s, er, pl.ds(0, LANE)]
                    a1 = p[1] * ebuf_v[s, er, pl.ds(LANE, LANE)]
                    for k in range(2, D // LANE, 2):
                        a0 = a0 + p[k] * ebuf_v[s, er,
                                                pl.ds(k * LANE, LANE)]
                        a1 = a1 + p[k + 1] * ebuf_v[s, er,
                                                    pl.ds((k + 1) * LANE,
                                                          LANE)]
                    sc = jnp.sum(a0 + a1)
                    svec = jnp.where(lanes == c, lax.broadcast(sc, (LANE,)),
                                     svec)
                sbuf_v[b, pl.ds(0, C)] = svec
                return carry2

            lax.fori_loop(0, SGRP, trow, 0)

            @pl.when(g + SBUF < NSG)
            def _():
                fire(g + SBUF, s)
        return carry

    lax.fori_loop(0, NSG // SBUF, outer, 0)
    pltpu.sync_copy(sbuf_v, out_hbm.at[pl.ds(w * RPW, RPW)])


def _score_call(cids_grp, proj, ent_emb):
    return pl.kernel(
        _score_body,
        out_type=jax.ShapeDtypeStruct((B, C), jnp.float32),
        mesh=_mesh,
        scratch_types=[
            pltpu.VMEM((NSG, 128), jnp.int32),
            pltpu.VMEM((SBUF, 128, D), jnp.float32),
            pltpu.VMEM((RPW, D), jnp.float32),
            pltpu.VMEM((RPW, C), jnp.float32),
        ] + [pltpu.SemaphoreType.DMA] * SBUF,
        compiler_params=_sc_params,
        name="yamada_score_sc",
    )(cids_grp, proj, ent_emb)


def kernel(word_emb, ent_emb, W, b, word_ids, cand_ent_ids):
    wids = word_ids.astype(jnp.int32)
    cids = cand_ent_ids.astype(jnp.int32)
    pad = (jnp.arange(B * (LP - L), dtype=jnp.int32).reshape(B, LP - L)
           * 9973) % (WV - 1) + 1
    wids_pad = jnp.concatenate([wids, pad], axis=1)
    wids_grp = wids_pad.reshape(B // G, GIDS)
    cids_grp = cids.reshape(B * C // 128, 128)

    pooled = _pool_call(wids_grp, word_emb).reshape(B, D)
    proj = _proj_call(pooled, wids, W, b.reshape(1, D))
    return _score_call(cids_grp, proj, ent_emb)


# speedup vs baseline: 1.0302x; 1.0302x over previous
import jax
import jax.numpy as jnp
from jax import lax
from jax.experimental import pallas as pl
from jax.experimental.pallas import tpu as pltpu
from jax.experimental.pallas import tpu_sc as plsc

B, L, C, D = 4096, 50, 16, 128
WV = 100000
LP = 52
NC, NS = 2, 16
NW = NC * NS
RPW = B // NW
G = 2
GIDS = G * LP
IDSW = 112
NGRP = RPW // G
SGRP = 16
NSG = RPW // SGRP
LANE = 16

_mesh = plsc.VectorSubcoreMesh(core_axis_name="c", subcore_axis_name="s")
_sc_params = pltpu.CompilerParams(needs_layout_passes=False)


def _worker_id():
    return lax.axis_index("s") * NC + lax.axis_index("c")


NBUF = 4
NOUT = NGRP // NBUF


def _pool_body(wids_hbm, wemb_hbm, out_hbm, ids_v, buf_v, pool_v, *sems):
    w = _worker_id()
    pltpu.sync_copy(wids_hbm.at[pl.ds(w * NGRP, NGRP)], ids_v)

    def fire(j, s):
        pltpu.async_copy(wemb_hbm.at[ids_v.at[j, pl.ds(0, GIDS)]],
                         buf_v.at[s], sems[s])

    for s in range(NBUF):
        fire(s, s)

    lanes = lax.iota(jnp.int32, LANE)

    def outer(jj, carry):
        for s in range(NBUF):
            j = jj * NBUF + s
            pltpu.make_async_copy(
                wemb_hbm.at[ids_v.at[j]], buf_v.at[s], sems[s]).wait()

            def accum(r, accs):
                return tuple(
                    accs[t * (D // LANE) + k]
                    + buf_v[s, t * L + r, pl.ds(k * LANE, LANE)]
                    for t in range(G) for k in range(D // LANE))

            init = tuple(buf_v[s, t * L, pl.ds(k * LANE, LANE)]
                         for t in range(G) for k in range(D // LANE))
            accs = lax.fori_loop(1, L, accum, init)

            def ones(col, lo, hi):
                nz = ids_v[j, pl.ds(col, LANE)] != 0
                if lo > 0 or hi < LANE:
                    nz = nz & (lanes >= lo) & (lanes < hi)
                return jnp.where(nz, 1.0, 0.0)

            c0 = (ones(0, 0, 16) + ones(16, 0, 16) + ones(32, 0, 16)
                  + ones(48, 0, 2))
            c1 = (ones(48, 2, 16) + ones(64, 0, 16) + ones(80, 0, 16)
                  + ones(96, 0, 4))
            inv = [1.0 / jnp.maximum(lax.broadcast(jnp.sum(c), (LANE,)), 1.0)
                   for c in (c0, c1)]
            for t in range(G):
                for k in range(D // LANE):
                    pool_v[pl.ds((j * G + t) * D + k * LANE, LANE)] = (
                        accs[t * (D // LANE) + k] * inv[t])

            @pl.when(j + NBUF < NGRP)
            def _():
                fire(j + NBUF, s)
        return carry

    lax.fori_loop(0, NOUT, outer, 0)
    pltpu.sync_copy(pool_v, out_hbm.at[pl.ds(w * RPW * D, RPW * D)])


def _pool_call(wids_grp, word_emb):
    return pl.kernel(
        _pool_body,
        out_type=jax.ShapeDtypeStruct((B * D,), jnp.float32),
        mesh=_mesh,
        scratch_types=[
            pltpu.VMEM((NGRP, IDSW), jnp.int32),
            pltpu.VMEM((NBUF, GIDS, D), jnp.float32),
            pltpu.VMEM((RPW * D,), jnp.float32),
        ] + [pltpu.SemaphoreType.DMA] * NBUF,
        compiler_params=_sc_params,
        name="yamada_pool_sc",
    )(wids_grp, word_emb)


def _proj_body(pooled_ref, w_ref, b_ref, out_ref):
    proj = lax.dot_general(pooled_ref[...], w_ref[...],
                           (((1,), (1,)), ((), ())),
                           preferred_element_type=jnp.float32)
    out_ref[...] = proj + b_ref[...]


def _proj_call(pooled, w, bias):
    return pl.pallas_call(
        _proj_body,
        out_shape=jax.ShapeDtypeStruct((B, D), jnp.float32),
        name="yamada_proj_tc",
    )(pooled, w, bias)


SBUF = 2


def _score_body(cids_hbm, proj_hbm, eemb_hbm, out_hbm, cid_v, ebuf_v, proj_v,
                sbuf_v, *sems):
    w = _worker_id()
    pltpu.sync_copy(cids_hbm.at[pl.ds(w * 2 * NSG, 2 * NSG)], cid_v)
    pltpu.sync_copy(proj_hbm.at[pl.ds(w * RPW, RPW)], proj_v)
    lanes = lax.iota(jnp.int32, LANE)

    def fire(g, s):
        pltpu.async_copy(eemb_hbm.at[cid_v.at[2 * g]],
                         ebuf_v.at[s, pl.ds(0, 128)], sems[s])
        pltpu.async_copy(eemb_hbm.at[cid_v.at[2 * g + 1]],
                         ebuf_v.at[s, pl.ds(128, 128)], sems[s])

    for s in range(SBUF):
        fire(s, s)

    def outer(gg, carry):
        for s in range(SBUF):
            g = gg * SBUF + s
            pltpu.make_async_copy(eemb_hbm.at[cid_v.at[2 * g]],
                                  ebuf_v.at[s, pl.ds(0, 128)], sems[s]).wait()
            pltpu.make_async_copy(eemb_hbm.at[cid_v.at[2 * g + 1]],
                                  ebuf_v.at[s, pl.ds(128, 128)], sems[s]).wait()

            def trow(t, carry2):
                b = g * SGRP + t
                p = [proj_v[b, pl.ds(k * LANE, LANE)]
                     for k in range(D // LANE)]
                svec = jnp.zeros((LANE,), jnp.float32)
                for c in range(C):
                    er = t * C + c
                    a0 = p[0] * ebuf_v[s, er, pl.ds(0, LANE)]
                    a1 = p[1] * ebuf_v[s, er, pl.ds(LANE, LANE)]
                    for k in range(2, D // LANE, 2):
                        a0 = a0 + p[k] * ebuf_v[s, er,
                                                pl.ds(k * LANE, LANE)]
                        a1 = a1 + p[k + 1] * ebuf_v[s, er,
                                                    pl.ds((k + 1) * LANE,
                                                          LANE)]
                    sc = jnp.sum(a0 + a1)
                    svec = jnp.where(lanes == c, lax.broadcast(sc, (LANE,)),
                                     svec)
                sbuf_v[b, pl.ds(0, C)] = svec
                return carry2

            lax.fori_loop(0, SGRP, trow, 0)

            @pl.when(g + SBUF < NSG)
            def _():
                fire(g + SBUF, s)
        return carry

    lax.fori_loop(0, NSG // SBUF, outer, 0)
    pltpu.sync_copy(sbuf_v, out_hbm.at[pl.ds(w * RPW, RPW)])


def _score_call(cids_grp, proj, ent_emb):
    return pl.kernel(
        _score_body,
        out_type=jax.ShapeDtypeStruct((B, C), jnp.float32),
        mesh=_mesh,
        scratch_types=[
            pltpu.VMEM((2 * NSG, 128), jnp.int32),
            pltpu.VMEM((SBUF, 2 * 128, D), jnp.float32),
            pltpu.VMEM((RPW, D), jnp.float32),
            pltpu.VMEM((RPW, C), jnp.float32),
        ] + [pltpu.SemaphoreType.DMA] * SBUF,
        compiler_params=_sc_params,
        name="yamada_score_sc",
    )(cids_grp, proj, ent_emb)


def kernel(word_emb, ent_emb, W, b, word_ids, cand_ent_ids):
    wids = word_ids.astype(jnp.int32)
    cids = cand_ent_ids.astype(jnp.int32)
    npad = IDSW - G * L
    pad = (jnp.arange((B // G) * npad, dtype=jnp.int32).reshape(B // G, npad)
           * 9973) % (WV - 1) + 1
    wids_grp = jnp.concatenate([wids.reshape(B // G, G * L), pad], axis=1)
    cids_grp = cids.reshape(B * C // 128, 128)

    pooled = _pool_call(wids_grp, word_emb).reshape(B, D)
    proj = _proj_call(pooled, W, b.reshape(1, D))
    return _score_call(cids_grp, proj, ent_emb)
